# Initial kernel scaffold; baseline (speedup 1.0000x reference)
import jax
import jax.numpy as jnp
from jax.experimental import pallas as pl


def kernel(token_values, W_val, b_val, type_table, side_table, slot_table, token_type_ids, token_side_ids, token_slot_ids):
    raise NotImplementedError("write your pallas kernel here")


# speedup vs baseline: 18.8980x; 18.8980x over previous
import functools

import jax
import jax.numpy as jnp
from jax.experimental import pallas as pl
from jax.experimental.pallas import tpu as pltpu


def _combine_body(ti_ref, si_ref, li_ref, tt_ref, st_ref, lt_ref, b_ref, c_ref):
    S = ti_ref.shape[0]

    def emb(ids_ref, table_ref):
        n = table_ref.shape[0]
        iota = jax.lax.broadcasted_iota(jnp.int32, (S, n), 1)
        oh = (ids_ref[...] == iota).astype(jnp.float32)
        return jax.lax.dot_general(
            oh, table_ref[...],
            dimension_numbers=(((1,), (0,)), ((), ())),
            preferred_element_type=jnp.float32)

    c_ref[...] = (emb(ti_ref, tt_ref) + emb(si_ref, st_ref)
                  + emb(li_ref, lt_ref) + b_ref[...])


def _fuse_body(tv_ref, w_ref, c_ref, out_ref):
    tv = tv_ref[...]
    w = w_ref[...][None]
    c = c_ref[...][None]
    out_ref[...] = tv * w + c


def kernel(token_values, W_val, b_val, type_table, side_table, slot_table,
           token_type_ids, token_side_ids, token_slot_ids):
    B, S = token_values.shape
    D = W_val.shape[0]

    ti = token_type_ids.reshape(S, 1)
    si = token_side_ids.reshape(S, 1)
    li = token_slot_ids.reshape(S, 1)
    w_row = W_val.reshape(1, D)
    b_row = b_val.reshape(1, D)
    tv3 = token_values.reshape(B, S, 1)

    combined = pl.pallas_call(
        _combine_body,
        out_shape=jax.ShapeDtypeStruct((S, D), jnp.float32),
    )(ti, si, li, type_table, side_table, slot_table, b_row)

    BB = 32
    fuse = pl.pallas_call(
        _fuse_body,
        grid=(B // BB,),
        in_specs=[
            pl.BlockSpec((BB, S, 1), lambda i: (i, 0, 0)),
            pl.BlockSpec((1, D), lambda i: (0, 0)),
            pl.BlockSpec((S, D), lambda i: (0, 0)),
        ],
        out_specs=pl.BlockSpec((BB, S, D), lambda i: (i, 0, 0)),
        out_shape=jax.ShapeDtypeStruct((B, S, D), jnp.float32),
        compiler_params=pltpu.CompilerParams(
            dimension_semantics=("parallel",)),
    )(tv3, w_row, combined)

    return fuse


# speedup vs baseline: 48.9867x; 2.5922x over previous
import functools

import jax
import jax.numpy as jnp
from jax.experimental import pallas as pl
from jax.experimental.pallas import tpu as pltpu


def _combine_body(ti_ref, si_ref, li_ref, tt_ref, st_ref, lt_ref, b_ref, c_ref):
    S = ti_ref.shape[0]

    def emb(ids_ref, table_ref):
        n = table_ref.shape[0]
        iota = jax.lax.broadcasted_iota(jnp.int32, (S, n), 1)
        oh = (ids_ref[...] == iota).astype(jnp.float32)
        return jax.lax.dot_general(
            oh, table_ref[...],
            dimension_numbers=(((1,), (0,)), ((), ())),
            preferred_element_type=jnp.float32)

    c_ref[...] = (emb(ti_ref, tt_ref) + emb(si_ref, st_ref)
                  + emb(li_ref, lt_ref) + b_ref[...])


def _fuse_body(tv_ref, w_ref, c_ref, out_ref):
    BB, S = tv_ref.shape
    tv = tv_ref[...].reshape(BB, S, 1)
    w = w_ref[...][None]
    c = c_ref[...][None]
    out_ref[...] = tv * w + c


def kernel(token_values, W_val, b_val, type_table, side_table, slot_table,
           token_type_ids, token_side_ids, token_slot_ids):
    B, S = token_values.shape
    D = W_val.shape[0]

    ti = token_type_ids.reshape(S, 1)
    si = token_side_ids.reshape(S, 1)
    li = token_slot_ids.reshape(S, 1)
    w_row = W_val.reshape(1, D)
    b_row = b_val.reshape(1, D)

    combined = pl.pallas_call(
        _combine_body,
        out_shape=jax.ShapeDtypeStruct((S, D), jnp.float32),
    )(ti, si, li, type_table, side_table, slot_table, b_row)

    BB = 32
    fuse = pl.pallas_call(
        _fuse_body,
        grid=(B // BB,),
        in_specs=[
            pl.BlockSpec((BB, S), lambda i: (i, 0)),
            pl.BlockSpec((1, D), lambda i: (0, 0)),
            pl.BlockSpec((S, D), lambda i: (0, 0)),
        ],
        out_specs=pl.BlockSpec((BB, S, D), lambda i: (i, 0, 0)),
        out_shape=jax.ShapeDtypeStruct((B, S, D), jnp.float32),
        compiler_params=pltpu.CompilerParams(
            dimension_semantics=("parallel",)),
    )(token_values, w_row, combined)

    return fuse


# speedup vs baseline: 57.6119x; 1.1761x over previous
import functools

import jax
import jax.numpy as jnp
from jax.experimental import pallas as pl
from jax.experimental.pallas import tpu as pltpu


def _combine_body(ti_ref, si_ref, li_ref, tt_ref, st_ref, lt_ref, b_ref, c_ref):
    S = ti_ref.shape[0]

    def emb(ids_ref, table_ref):
        n = table_ref.shape[0]
        iota = jax.lax.broadcasted_iota(jnp.int32, (S, n), 1)
        oh = (ids_ref[...] == iota).astype(jnp.float32)
        return jax.lax.dot_general(
            oh, table_ref[...],
            dimension_numbers=(((1,), (0,)), ((), ())),
            preferred_element_type=jnp.float32)

    c_ref[...] = (emb(ti_ref, tt_ref) + emb(si_ref, st_ref)
                  + emb(li_ref, lt_ref) + b_ref[...])


def _fuse_body(tv_ref, w_ref, c_ref, out_ref):
    BB, S = tv_ref.shape
    tv = tv_ref[...].reshape(BB, S, 1)
    w = w_ref[...][None]
    c = c_ref[...][None]
    out_ref[...] = tv * w + c


def kernel(token_values, W_val, b_val, type_table, side_table, slot_table,
           token_type_ids, token_side_ids, token_slot_ids):
    B, S = token_values.shape
    D = W_val.shape[0]

    ti = token_type_ids.reshape(S, 1)
    si = token_side_ids.reshape(S, 1)
    li = token_slot_ids.reshape(S, 1)
    w_row = W_val.reshape(1, D)
    b_row = b_val.reshape(1, D)

    combined = pl.pallas_call(
        _combine_body,
        out_shape=jax.ShapeDtypeStruct((S, D), jnp.float32),
    )(ti, si, li, type_table, side_table, slot_table, b_row)

    BB = 64
    fuse = pl.pallas_call(
        _fuse_body,
        grid=(B // BB,),
        in_specs=[
            pl.BlockSpec((BB, S), lambda i: (i, 0)),
            pl.BlockSpec((1, D), lambda i: (0, 0)),
            pl.BlockSpec((S, D), lambda i: (0, 0)),
        ],
        out_specs=pl.BlockSpec((BB, S, D), lambda i: (i, 0, 0)),
        out_shape=jax.ShapeDtypeStruct((B, S, D), jnp.float32),
        compiler_params=pltpu.CompilerParams(
            dimension_semantics=("parallel",)),
    )(token_values, w_row, combined)

    return fuse


# speedup vs baseline: 61.3221x; 1.0644x over previous
import functools

import jax
import jax.numpy as jnp
from jax.experimental import pallas as pl
from jax.experimental.pallas import tpu as pltpu


def _combine_body(ti_ref, si_ref, li_ref, tt_ref, st_ref, lt_ref, b_ref, c_ref):
    S = ti_ref.shape[0]

    def emb(ids_ref, table_ref):
        n = table_ref.shape[0]
        iota = jax.lax.broadcasted_iota(jnp.int32, (S, n), 1)
        oh = (ids_ref[...] == iota).astype(jnp.float32)
        return jax.lax.dot_general(
            oh, table_ref[...],
            dimension_numbers=(((1,), (0,)), ((), ())),
            preferred_element_type=jnp.float32)

    c_ref[...] = (emb(ti_ref, tt_ref) + emb(si_ref, st_ref)
                  + emb(li_ref, lt_ref) + b_ref[...])


def _fuse_body(tv_ref, w_ref, c_ref, out_ref):
    BB, S = tv_ref.shape
    tv = tv_ref[...].reshape(BB, S, 1)
    w = w_ref[...][None]
    c = c_ref[...][None]
    out_ref[...] = tv * w + c


def kernel(token_values, W_val, b_val, type_table, side_table, slot_table,
           token_type_ids, token_side_ids, token_slot_ids):
    B, S = token_values.shape
    D = W_val.shape[0]

    ti = token_type_ids.reshape(S, 1)
    si = token_side_ids.reshape(S, 1)
    li = token_slot_ids.reshape(S, 1)
    w_row = W_val.reshape(1, D)
    b_row = b_val.reshape(1, D)

    combined = pl.pallas_call(
        _combine_body,
        out_shape=jax.ShapeDtypeStruct((S, D), jnp.float32),
    )(ti, si, li, type_table, side_table, slot_table, b_row)

    BB = 128
    fuse = pl.pallas_call(
        _fuse_body,
        grid=(B // BB,),
        in_specs=[
            pl.BlockSpec((BB, S), lambda i: (i, 0)),
            pl.BlockSpec((1, D), lambda i: (0, 0)),
            pl.BlockSpec((S, D), lambda i: (0, 0)),
        ],
        out_specs=pl.BlockSpec((BB, S, D), lambda i: (i, 0, 0)),
        out_shape=jax.ShapeDtypeStruct((B, S, D), jnp.float32),
        compiler_params=pltpu.CompilerParams(
            dimension_semantics=("parallel",)),
    )(token_values, w_row, combined)

    return fuse
